# baseline (speedup 1.0000x reference)
import functools

import jax
import jax.numpy as jnp
from jax import lax
from jax.experimental import pallas as pl
from jax.experimental.pallas import tpu as pltpu
from jax.experimental.pallas import tpu_sc as plsc

B = 128
L = 1024
D = 96
KEEP = 512
ROW_IN = L + 1
ROW_OUT = KEEP + 1
PAD_OUT = 520
NC, NS = 2, 16
NW = NC * NS
ROWS_PER_W = B // NW
GCHUNK = 104
NGCHUNK = PAD_OUT // GCHUNK

_MININT_PY = -2**31


def _monotone_key(s):
    return s ^ (lax.shift_right_arithmetic(s, 31) & jnp.int32(0x7FFFFFFF))


def _select_body(noise_ref, thr_ref, need_ref):
    s = lax.bitcast_convert_type(noise_ref[...], jnp.int32)
    m = _monotone_key(s)
    u = m ^ jnp.int32(_MININT_PY)
    p = jnp.zeros((B, 1), jnp.int32)
    kk = jnp.full((B, 1), KEEP - 1, jnp.int32)
    for bit in range(31, -1, -1):
        ub = lax.shift_right_logical(u, bit)
        pb = lax.shift_right_logical(p, bit)
        cnt = jnp.sum((ub == pb).astype(jnp.int32), axis=1, keepdims=True)
        take = kk >= cnt
        bitval = jnp.int32(_MININT_PY if bit == 31 else 1 << bit)
        p = jnp.where(take, p | bitval, p)
        kk = jnp.where(take, kk - cnt, kk)
    t_m = p ^ jnp.int32(_MININT_PY)
    cnt_less = jnp.sum((m < t_m).astype(jnp.int32), axis=1, keepdims=True)
    need = KEEP - cnt_less
    thr_ref[...] = jnp.broadcast_to(t_m[:, :, None], (B, 1, NS))
    need_ref[...] = jnp.broadcast_to(need[:, :, None], (B, 1, NS))


def _tc_select(noise):
    return pl.pallas_call(
        _select_body,
        out_shape=[
            jax.ShapeDtypeStruct((B, 1, NS), jnp.int32),
            jax.ShapeDtypeStruct((B, 1, NS), jnp.int32),
        ],
    )(noise)


def _sc_body(noise_hbm, thr_hbm, need_hbm, x_hbm, out_hbm,
             noise_v, thr_v, need_v, idx_v, rows_v, sem):
    wid = lax.axis_index("s") * NC + lax.axis_index("c")
    lanes = lax.iota(jnp.int32, NS)
    for r in range(ROWS_PER_W):
        b = wid * ROWS_PER_W + r
        pltpu.sync_copy(noise_hbm.at[b], noise_v)
        pltpu.sync_copy(thr_hbm.at[b], thr_v)
        pltpu.sync_copy(need_hbm.at[b], need_v)
        t_vec = thr_v[0]
        need_vec = need_v[0]
        prefix_gid = b * ROW_IN
        pad_slots = jnp.where(lanes == 0, 0, KEEP + lanes)
        plsc.store_scatter(idx_v, [pad_slots],
                           jnp.full((NS,), 0, jnp.int32) + prefix_gid,
                           mask=lanes < 8)

        def chunk(k, carry):
            seq, ties = carry
            v = noise_v[k >> 3, pl.ds((k & 7) * NS, NS)]
            m = _monotone_key(plsc.bitcast(v, jnp.int32))
            less = m < t_vec
            eq = m == t_vec
            eq_i = jnp.where(eq, 1, 0).astype(jnp.int32)
            cum_eq = plsc.cumsum(eq_i)
            tie_rank = (ties + cum_eq) - eq_i
            keep = jnp.logical_or(less, jnp.logical_and(eq, tie_rank < need_vec))
            keep_i = jnp.where(keep, 1, 0).astype(jnp.int32)
            slot = seq + plsc.cumsum(keep_i)
            gid = (prefix_gid + 1 + k * NS) + lanes
            plsc.store_scatter(idx_v, [slot], gid, mask=keep)
            return seq + jnp.sum(keep_i), ties + jnp.sum(eq_i)

        lax.fori_loop(0, L // NS, chunk, (jnp.int32(0), jnp.int32(0)))

        handles = [
            pltpu.async_copy(x_hbm.at[idx_v.at[pl.ds(j * GCHUNK, GCHUNK)]],
                             rows_v.at[pl.ds(j * GCHUNK, GCHUNK)], sem)
            for j in range(NGCHUNK)
        ]
        for h in handles:
            h.wait()
        pltpu.sync_copy(rows_v.at[pl.ds(0, ROW_OUT)], out_hbm.at[b])


def _sc_gather(noise, thr, need, x_flat):
    mesh = plsc.VectorSubcoreMesh(core_axis_name="c", subcore_axis_name="s")
    k = functools.partial(
        pl.kernel,
        mesh=mesh,
        out_type=jax.ShapeDtypeStruct((B, ROW_OUT, D), jnp.float32),
        scratch_types=[
            pltpu.VMEM((L // 128, 128), jnp.float32),
            pltpu.VMEM((1, NS), jnp.int32),
            pltpu.VMEM((1, NS), jnp.int32),
            pltpu.VMEM((PAD_OUT,), jnp.int32),
            pltpu.VMEM((PAD_OUT, D), jnp.float32),
            pltpu.SemaphoreType.DMA,
        ],
        compiler_params=pltpu.CompilerParams(
            needs_layout_passes=False, use_tc_tiling_on_sc=False),
    )(_sc_body)
    return k(noise, thr, need, x_flat)


def kernel(x):
    noise = jax.random.normal(jax.random.key(1), (B, L), dtype=jnp.float32)
    thr, need = _tc_select(noise)
    x_flat = x.reshape(B * ROW_IN, D)
    return _sc_gather(noise.reshape(B, L // 128, 128), thr, need, x_flat)


# speedup vs baseline: 1.0008x; 1.0008x over previous
import functools

import jax
import jax.numpy as jnp
from jax import lax
from jax.experimental import pallas as pl
from jax.experimental.pallas import tpu as pltpu
from jax.experimental.pallas import tpu_sc as plsc

B = 128
L = 1024
D = 96
KEEP = 512
ROW_IN = L + 1
ROW_OUT = KEEP + 1
PAD_OUT = 520
NC, NS = 2, 16
NW = NC * NS
ROWS_PER_W = B // NW
GCHUNK = 104
NGCHUNK = PAD_OUT // GCHUNK

_MININT_PY = -2**31


def _monotone_key(s):
    return s ^ (lax.shift_right_arithmetic(s, 31) & jnp.int32(0x7FFFFFFF))


def _select_body(noise_ref, thr_ref, need_ref):
    s = lax.bitcast_convert_type(noise_ref[...], jnp.int32)
    m = _monotone_key(s)
    u = m ^ jnp.int32(_MININT_PY)
    p = jnp.zeros((B, 1), jnp.int32)
    kk = jnp.full((B, 1), KEEP - 1, jnp.int32)
    for bit in range(31, -1, -1):
        ub = lax.shift_right_logical(u, bit)
        pb = lax.shift_right_logical(p, bit)
        cnt = jnp.sum((ub == pb).astype(jnp.int32), axis=1, keepdims=True)
        take = kk >= cnt
        bitval = jnp.int32(_MININT_PY if bit == 31 else 1 << bit)
        p = jnp.where(take, p | bitval, p)
        kk = jnp.where(take, kk - cnt, kk)
    t_m = p ^ jnp.int32(_MININT_PY)
    cnt_less = jnp.sum((m < t_m).astype(jnp.int32), axis=1, keepdims=True)
    need = KEEP - cnt_less
    thr_ref[...] = jnp.broadcast_to(t_m[:, :, None], (B, 1, NS))
    need_ref[...] = jnp.broadcast_to(need[:, :, None], (B, 1, NS))


def _tc_select(noise):
    return pl.pallas_call(
        _select_body,
        out_shape=[
            jax.ShapeDtypeStruct((B, 1, NS), jnp.int32),
            jax.ShapeDtypeStruct((B, 1, NS), jnp.int32),
        ],
    )(noise)


def _sc_body(noise_hbm, thr_hbm, need_hbm, x_hbm, out_hbm,
             noise_v, thr_v, need_v, idx_v, rows_v, sem):
    wid = lax.axis_index("s") * NC + lax.axis_index("c")
    lanes = lax.iota(jnp.int32, NS)
    for r in range(ROWS_PER_W):
        b = wid * ROWS_PER_W + r
        pltpu.sync_copy(noise_hbm.at[b], noise_v)
        pltpu.sync_copy(thr_hbm.at[b], thr_v)
        pltpu.sync_copy(need_hbm.at[b], need_v)
        t_vec = thr_v[0]
        need_vec = need_v[0]
        pad_slots = jnp.where(lanes == 0, 0, KEEP + lanes)
        plsc.store_scatter(idx_v, [pad_slots],
                           jnp.zeros((NS,), jnp.int32), mask=lanes < 8)

        def chunk(k, carry):
            seq, ties = carry
            v = noise_v[k >> 3, pl.ds((k & 7) * NS, NS)]
            m = _monotone_key(plsc.bitcast(v, jnp.int32))
            less = m < t_vec
            eq = m == t_vec
            eq_i = jnp.where(eq, 1, 0).astype(jnp.int32)
            cum_eq = plsc.cumsum(eq_i)
            tie_rank = (ties + cum_eq) - eq_i
            keep = jnp.logical_or(less, jnp.logical_and(eq, tie_rank < need_vec))
            keep_i = jnp.where(keep, 1, 0).astype(jnp.int32)
            slot = seq + plsc.cumsum(keep_i)
            gid = (1 + k * NS) + lanes
            plsc.store_scatter(idx_v, [slot], gid, mask=keep)
            return seq + jnp.sum(keep_i), ties + jnp.sum(eq_i)

        lax.fori_loop(0, L // NS, chunk, (jnp.int32(0), jnp.int32(0)))

        handles = [
            pltpu.async_copy(x_hbm.at[b].at[idx_v.at[pl.ds(j * GCHUNK, GCHUNK)]],
                             rows_v.at[pl.ds(j * GCHUNK, GCHUNK)], sem)
            for j in range(NGCHUNK)
        ]
        for h in handles:
            h.wait()
        pltpu.sync_copy(rows_v.at[pl.ds(0, ROW_OUT)], out_hbm.at[b])


def _sc_gather(noise, thr, need, x_flat):
    mesh = plsc.VectorSubcoreMesh(core_axis_name="c", subcore_axis_name="s")
    k = functools.partial(
        pl.kernel,
        mesh=mesh,
        out_type=jax.ShapeDtypeStruct((B, ROW_OUT, D), jnp.float32),
        scratch_types=[
            pltpu.VMEM((L // 128, 128), jnp.float32),
            pltpu.VMEM((1, NS), jnp.int32),
            pltpu.VMEM((1, NS), jnp.int32),
            pltpu.VMEM((PAD_OUT,), jnp.int32),
            pltpu.VMEM((PAD_OUT, D), jnp.float32),
            pltpu.SemaphoreType.DMA,
        ],
        compiler_params=pltpu.CompilerParams(
            needs_layout_passes=False, use_tc_tiling_on_sc=False),
    )(_sc_body)
    return k(noise, thr, need, x_flat)


def kernel(x):
    noise = jax.random.normal(jax.random.key(1), (B, L), dtype=jnp.float32)
    thr, need = _tc_select(noise)
    return _sc_gather(noise.reshape(B, L // 128, 128), thr, need, x)
